# baseline (speedup 1.0000x reference)
import functools

import jax
import jax.numpy as jnp
from jax import lax
from jax.experimental import pallas as pl
from jax.experimental.pallas import tpu as pltpu
from jax.experimental.pallas import tpu_sc as plsc

EPS = 1e-12

NC = 2
NS = 16
NW = NC * NS

IROW = 128
G2 = 1
NBUF = 2


def _sc_gather_pack(table, idx2d, n_tokens, d):
    n_rows = idx2d.shape[0]
    rows_per_tile = n_rows // NW
    chunk = G2 * IROW
    chunks = rows_per_tile // G2
    pairs = chunks // NBUF
    h = d // 2

    mesh = plsc.VectorSubcoreMesh(core_axis_name="c", subcore_axis_name="s")

    @functools.partial(
        pl.kernel,
        out_type=jax.ShapeDtypeStruct((n_tokens, h), jnp.int32),
        mesh=mesh,
        scratch_types=[
            pltpu.VMEM((rows_per_tile, IROW), jnp.int32),
            pltpu.VMEM((NBUF, chunk, d), jnp.int32),
            pltpu.VMEM((NBUF, chunk, h), jnp.int32),
            pltpu.SemaphoreType.DMA,
            pltpu.SemaphoreType.DMA,
            pltpu.SemaphoreType.DMA,
            pltpu.SemaphoreType.DMA,
        ],
    )
    def k(table_hbm, idx_hbm, out_hbm, idx_v, rows_v, pk_v, g0, g1, s0, s1):
        wid = lax.axis_index("s") * NC + lax.axis_index("c")
        row0 = wid * rows_per_tile
        tok0 = row0 * IROW
        gsem = (g0, g1)
        ssem = (s0, s1)

        pltpu.sync_copy(idx_hbm.at[pl.ds(row0, rows_per_tile)], idx_v)

        def fire(c, b):
            for j in range(G2):
                pltpu.async_copy(
                    table_hbm.at[idx_v.at[c * G2 + j]],
                    rows_v.at[b].at[pl.ds(j * IROW, IROW)],
                    gsem[b],
                )

        def drain_gather(b):
            for _ in range(G2):
                pltpu.make_async_copy(
                    table_hbm.at[pl.ds(0, IROW)],
                    rows_v.at[b].at[pl.ds(0, IROW)],
                    gsem[b],
                ).wait()

        def rtne16(bits):
            rnd = lax.shift_right_logical(bits, 16) & jnp.int32(1)
            return lax.shift_right_logical(bits + jnp.int32(0x7FFF) + rnd, 16)

        def pack_chunk(b):
            def tok_body(tok, _):
                for j in range(0, h, 16):
                    lo = rtne16(rows_v[b, tok, pl.ds(j, 16)])
                    hi = rtne16(rows_v[b, tok, pl.ds(h + j, 16)])
                    pk_v[b, tok, pl.ds(j, 16)] = (hi << 16) | lo
                return ()
            lax.fori_loop(0, chunk, tok_body, (), unroll=False)

        def scatter(c, b):
            pltpu.async_copy(
                pk_v.at[b],
                out_hbm.at[pl.ds(tok0 + c * chunk, chunk)],
                ssem[b],
            )

        def drain_scatter(b):
            pltpu.make_async_copy(
                pk_v.at[b], out_hbm.at[pl.ds(0, chunk)], ssem[b]
            ).wait()

        fire(0, 0)

        def body(i, _):
            c0 = i * NBUF
            drain_gather(0)
            fire(c0 + 1, 1)
            pack_chunk(0)

            @pl.when(i > 0)
            def _():
                drain_scatter(0)

            scatter(c0, 0)

            drain_gather(1)

            @pl.when(i < pairs - 1)
            def _():
                fire(c0 + 2, 0)

            pack_chunk(1)

            @pl.when(i > 0)
            def _():
                drain_scatter(1)

            scatter(c0 + 1, 1)
            return ()

        lax.fori_loop(0, pairs, body, (), unroll=False)
        drain_scatter(0)
        drain_scatter(1)

    return k(table, idx2d)


def _tc_type_ln_slab(buf, y_k, tids3_k, tt_pad, gamma2d, beta2d,
                     tb, d, n_tokens, block0):
    slab_blocks = y_k.shape[0] // tb

    def body(*refs):
        if buf is None:
            ids_ref, y_ref, tt_ref, g_ref, b_ref, o_ref = refs
        else:
            _, ids_ref, y_ref, tt_ref, g_ref, b_ref, o_ref = refs
        ids = ids_ref[0, 0, :]
        oh = (ids[:, None] == lax.broadcasted_iota(jnp.int32, (tb, d), 1))
        temb = jax.lax.dot(
            oh.astype(jnp.float32), tt_ref[...],
            precision=jax.lax.Precision.DEFAULT,
        )
        w = y_ref[...]
        x_lo = lax.bitcast_convert_type(w << 16, jnp.float32)
        x_hi = lax.bitcast_convert_type(w & jnp.int32(-65536), jnp.float32)
        x = jnp.concatenate([x_lo, x_hi], axis=-1) + temb
        jn = jnp.full((d, d), 1.0 / d, jnp.float32)
        mean = jax.lax.dot(x, jn, precision=jax.lax.Precision.DEFAULT)
        c = x - mean
        var = jax.lax.dot(c * c, jn, precision=jax.lax.Precision.DEFAULT)
        r = jax.lax.rsqrt(var + EPS)
        o_ref[...] = (c * r) * g_ref[...] + b_ref[...]

    in_specs = [
        pl.BlockSpec((1, 1, tb), lambda i: (i, 0, 0)),
        pl.BlockSpec((tb, d // 2), lambda i: (i, 0)),
        pl.BlockSpec((d, d), lambda i: (0, 0)),
        pl.BlockSpec((1, d), lambda i: (0, 0)),
        pl.BlockSpec((1, d), lambda i: (0, 0)),
    ]
    args = [tids3_k, y_k, tt_pad, gamma2d, beta2d]
    aliases = {}
    if buf is not None:
        in_specs = [pl.BlockSpec(memory_space=pl.ANY)] + in_specs
        args = [buf] + args
        aliases = {0: 0}

    return pl.pallas_call(
        body,
        grid=(slab_blocks,),
        in_specs=in_specs,
        out_specs=pl.BlockSpec((tb, d), lambda i: (block0 + i, 0)),
        out_shape=jax.ShapeDtypeStruct((n_tokens, d), jnp.float32),
        input_output_aliases=aliases,
    )(*args)


K_SLABS = 5


def kernel(input_ids, type_ids, input_table, type_table, ln_gamma, ln_beta):
    b, s = input_ids.shape
    d = input_table.shape[1]
    n_tokens = b * s
    tb = 4096

    slab = n_tokens // K_SLABS
    idx2d = input_ids.reshape(n_tokens // IROW, IROW)
    tids3 = type_ids.reshape(n_tokens // tb, 1, tb)

    tv = type_table.shape[0]
    tt_pad = jnp.zeros((d, d), jnp.float32).at[:tv].set(type_table)
    g2 = ln_gamma.reshape(1, d)
    b2 = ln_beta.reshape(1, d)

    srows = slab // IROW
    sblocks = slab // tb
    tbl_bits = lax.bitcast_convert_type(input_table, jnp.int32)
    ys = [
        _sc_gather_pack(
            tbl_bits,
            lax.slice_in_dim(idx2d, k * srows, (k + 1) * srows), slab, d)
        for k in range(K_SLABS)
    ]
    buf = None
    for k in range(K_SLABS):
        buf = _tc_type_ln_slab(
            buf, ys[k],
            lax.slice_in_dim(tids3, k * sblocks, (k + 1) * sblocks),
            tt_pad, g2, b2, tb, d, n_tokens, k * sblocks,
        )
    return buf.reshape(b, s, d)


# speedup vs baseline: 1.0007x; 1.0007x over previous
import functools

import jax
import jax.numpy as jnp
from jax import lax
from jax.experimental import pallas as pl
from jax.experimental.pallas import tpu as pltpu
from jax.experimental.pallas import tpu_sc as plsc

EPS = 1e-12

NC = 2
NS = 16
NW = NC * NS

IROW = 128
G2 = 1
NBUF = 2


def _sc_gather_pack(table, idx2d, n_tokens, d):
    n_rows = idx2d.shape[0]
    rows_per_tile = n_rows // NW
    chunk = G2 * IROW
    chunks = rows_per_tile // G2
    pairs = chunks // NBUF
    h = d // 2

    mesh = plsc.VectorSubcoreMesh(core_axis_name="c", subcore_axis_name="s")

    @functools.partial(
        pl.kernel,
        out_type=jax.ShapeDtypeStruct((n_tokens, h), jnp.int32),
        mesh=mesh,
        scratch_types=[
            pltpu.VMEM((NBUF, G2, IROW), jnp.int32),
            pltpu.VMEM((NBUF, chunk, d), jnp.int32),
            pltpu.VMEM((NBUF, chunk, h), jnp.int32),
            pltpu.SemaphoreType.DMA,
            pltpu.SemaphoreType.DMA,
            pltpu.SemaphoreType.DMA,
            pltpu.SemaphoreType.DMA,
            pltpu.SemaphoreType.DMA,
            pltpu.SemaphoreType.DMA,
        ],
    )
    def k(table_hbm, idx_hbm, out_hbm, idx_v, rows_v, pk_v,
          g0, g1, s0, s1, i0, i1):
        wid = lax.axis_index("s") * NC + lax.axis_index("c")
        row0 = wid * rows_per_tile
        tok0 = row0 * IROW
        gsem = (g0, g1)
        ssem = (s0, s1)
        isem = (i0, i1)

        def fire_idx(c, b):
            pltpu.async_copy(
                idx_hbm.at[pl.ds(row0 + c * G2, G2)], idx_v.at[b], isem[b]
            )

        def drain_idx(b):
            pltpu.make_async_copy(
                idx_hbm.at[pl.ds(row0, G2)], idx_v.at[b], isem[b]
            ).wait()

        def fire(c, b):
            for j in range(G2):
                pltpu.async_copy(
                    table_hbm.at[idx_v.at[b].at[j]],
                    rows_v.at[b].at[pl.ds(j * IROW, IROW)],
                    gsem[b],
                )

        def drain_gather(b):
            for _ in range(G2):
                pltpu.make_async_copy(
                    table_hbm.at[pl.ds(0, IROW)],
                    rows_v.at[b].at[pl.ds(0, IROW)],
                    gsem[b],
                ).wait()

        def rtne16(bits):
            rnd = lax.shift_right_logical(bits, 16) & jnp.int32(1)
            return lax.shift_right_logical(bits + jnp.int32(0x7FFF) + rnd, 16)

        def pack_chunk(b):
            @plsc.parallel_loop(0, chunk, unroll=4)
            def _(tok):
                for j in range(0, h, 16):
                    lo = rtne16(rows_v[b, tok, pl.ds(j, 16)])
                    hi = rtne16(rows_v[b, tok, pl.ds(h + j, 16)])
                    pk_v[b, tok, pl.ds(j, 16)] = (hi << 16) | lo

        def scatter(c, b):
            pltpu.async_copy(
                pk_v.at[b],
                out_hbm.at[pl.ds(tok0 + c * chunk, chunk)],
                ssem[b],
            )

        def drain_scatter(b):
            pltpu.make_async_copy(
                pk_v.at[b], out_hbm.at[pl.ds(0, chunk)], ssem[b]
            ).wait()

        fire_idx(0, 0)
        fire_idx(1, 1)
        drain_idx(0)
        fire(0, 0)

        def body(i, _):
            c0 = i * NBUF
            drain_gather(0)
            drain_idx(1)
            fire(c0 + 1, 1)
            fire_idx(c0 + 2, 0)
            pack_chunk(0)

            @pl.when(i > 0)
            def _():
                drain_scatter(0)

            scatter(c0, 0)

            drain_gather(1)

            @pl.when(i < pairs - 1)
            def _():
                drain_idx(0)
                fire(c0 + 2, 0)
                fire_idx(c0 + 3, 1)

            pack_chunk(1)

            @pl.when(i > 0)
            def _():
                drain_scatter(1)

            scatter(c0 + 1, 1)
            return ()

        lax.fori_loop(0, pairs, body, (), unroll=False)
        drain_scatter(0)
        drain_scatter(1)
        drain_idx(0)

    return k(table, idx2d)


def _tc_type_ln_slab(buf, y_k, tids3_k, tt_pad, gamma2d, beta2d,
                     tb, d, n_tokens, block0):
    slab_blocks = y_k.shape[0] // tb

    def body(*refs):
        if buf is None:
            ids_ref, y_ref, tt_ref, g_ref, b_ref, o_ref = refs
        else:
            _, ids_ref, y_ref, tt_ref, g_ref, b_ref, o_ref = refs
        ids = ids_ref[0, 0, :]
        oh = (ids[:, None] == lax.broadcasted_iota(jnp.int32, (tb, d), 1))
        temb = jax.lax.dot(
            oh.astype(jnp.float32), tt_ref[...],
            precision=jax.lax.Precision.DEFAULT,
        )
        w = y_ref[...]
        x_lo = lax.bitcast_convert_type(w << 16, jnp.float32)
        x_hi = lax.bitcast_convert_type(w & jnp.int32(-65536), jnp.float32)
        x = jnp.concatenate([x_lo, x_hi], axis=-1) + temb
        jn = jnp.full((d, d), 1.0 / d, jnp.float32)
        mean = jax.lax.dot(x, jn, precision=jax.lax.Precision.DEFAULT)
        c = x - mean
        var = jax.lax.dot(c * c, jn, precision=jax.lax.Precision.DEFAULT)
        r = jax.lax.rsqrt(var + EPS)
        o_ref[...] = (c * r) * g_ref[...] + b_ref[...]

    in_specs = [
        pl.BlockSpec((1, 1, tb), lambda i: (i, 0, 0)),
        pl.BlockSpec((tb, d // 2), lambda i: (i, 0)),
        pl.BlockSpec((d, d), lambda i: (0, 0)),
        pl.BlockSpec((1, d), lambda i: (0, 0)),
        pl.BlockSpec((1, d), lambda i: (0, 0)),
    ]
    args = [tids3_k, y_k, tt_pad, gamma2d, beta2d]
    aliases = {}
    if buf is not None:
        in_specs = [pl.BlockSpec(memory_space=pl.ANY)] + in_specs
        args = [buf] + args
        aliases = {0: 0}

    return pl.pallas_call(
        body,
        grid=(slab_blocks,),
        in_specs=in_specs,
        out_specs=pl.BlockSpec((tb, d), lambda i: (block0 + i, 0)),
        out_shape=jax.ShapeDtypeStruct((n_tokens, d), jnp.float32),
        input_output_aliases=aliases,
    )(*args)


K_SLABS = 5


def kernel(input_ids, type_ids, input_table, type_table, ln_gamma, ln_beta):
    b, s = input_ids.shape
    d = input_table.shape[1]
    n_tokens = b * s
    tb = 4096

    slab = n_tokens // K_SLABS
    idx2d = input_ids.reshape(n_tokens // IROW, IROW)
    tids3 = type_ids.reshape(n_tokens // tb, 1, tb)

    tv = type_table.shape[0]
    tt_pad = jnp.zeros((d, d), jnp.float32).at[:tv].set(type_table)
    g2 = ln_gamma.reshape(1, d)
    b2 = ln_beta.reshape(1, d)

    srows = slab // IROW
    sblocks = slab // tb
    tbl_bits = lax.bitcast_convert_type(input_table, jnp.int32)
    ys = [
        _sc_gather_pack(
            tbl_bits,
            lax.slice_in_dim(idx2d, k * srows, (k + 1) * srows), slab, d)
        for k in range(K_SLABS)
    ]
    buf = None
    for k in range(K_SLABS):
        buf = _tc_type_ln_slab(
            buf, ys[k],
            lax.slice_in_dim(tids3, k * sblocks, (k + 1) * sblocks),
            tt_pad, g2, b2, tb, d, n_tokens, k * sblocks,
        )
    return buf.reshape(b, s, d)


# speedup vs baseline: 1.0310x; 1.0303x over previous
import functools

import jax
import jax.numpy as jnp
from jax import lax
from jax.experimental import pallas as pl
from jax.experimental.pallas import tpu as pltpu
from jax.experimental.pallas import tpu_sc as plsc

EPS = 1e-12

NC = 2
NS = 16
NW = NC * NS

IROW = 128
G2 = 2
NBUF = 2


def _sc_gather(table, idx2d, n_tokens, d):
    n_rows = idx2d.shape[0]
    rows_per_tile = n_rows // NW
    chunk = G2 * IROW
    chunks = rows_per_tile // G2
    pairs = chunks // NBUF

    mesh = plsc.VectorSubcoreMesh(core_axis_name="c", subcore_axis_name="s")

    @functools.partial(
        pl.kernel,
        out_type=jax.ShapeDtypeStruct((n_tokens, d), table.dtype),
        mesh=mesh,
        scratch_types=[
            pltpu.VMEM((rows_per_tile, IROW), jnp.int32),
            pltpu.VMEM((NBUF, chunk, d), table.dtype),
            pltpu.SemaphoreType.DMA,
            pltpu.SemaphoreType.DMA,
            pltpu.SemaphoreType.DMA,
            pltpu.SemaphoreType.DMA,
        ],
    )
    def k(table_hbm, idx_hbm, out_hbm, idx_v, rows_v, g0, g1, s0, s1):
        wid = lax.axis_index("s") * NC + lax.axis_index("c")
        row0 = wid * rows_per_tile
        tok0 = row0 * IROW
        gsem = (g0, g1)
        ssem = (s0, s1)

        pltpu.sync_copy(idx_hbm.at[pl.ds(row0, rows_per_tile)], idx_v)

        def fire(c, b):
            for j in range(G2):
                pltpu.async_copy(
                    table_hbm.at[idx_v.at[c * G2 + j]],
                    rows_v.at[b].at[pl.ds(j * IROW, IROW)],
                    gsem[b],
                )

        def drain_gather(b):
            pltpu.make_async_copy(
                out_hbm.at[pl.ds(0, chunk)], rows_v.at[b], gsem[b]
            ).wait()

        def scatter(c, b):
            pltpu.async_copy(
                rows_v.at[b],
                out_hbm.at[pl.ds(tok0 + c * chunk, chunk)],
                ssem[b],
            )

        def drain_scatter(b):
            pltpu.make_async_copy(
                rows_v.at[b], out_hbm.at[pl.ds(0, chunk)], ssem[b]
            ).wait()

        fire(0, 0)

        def body(i, _):
            c0 = i * NBUF
            drain_gather(0)
            scatter(c0, 0)

            @pl.when(i > 0)
            def _():
                drain_scatter(1)

            fire(c0 + 1, 1)

            drain_gather(1)
            scatter(c0 + 1, 1)
            drain_scatter(0)

            @pl.when(i < pairs - 1)
            def _():
                fire(c0 + 2, 0)

            return ()

        lax.fori_loop(0, pairs, body, (), unroll=False)
        drain_scatter(1)

    return k(table, idx2d)


def _tc_type_ln_slab(buf, y_k, tids3_k, tt_pad, gamma2d, beta2d,
                     tb, d, n_tokens, block0):
    slab_blocks = y_k.shape[0] // tb

    def body(*refs):
        if buf is None:
            ids_ref, y_ref, tt_ref, g_ref, b_ref, o_ref = refs
        else:
            _, ids_ref, y_ref, tt_ref, g_ref, b_ref, o_ref = refs
        ids = ids_ref[0, 0, :]
        oh = (ids[:, None] == lax.broadcasted_iota(jnp.int32, (tb, d), 1))
        temb = jax.lax.dot(
            oh.astype(jnp.float32), tt_ref[...],
            precision=jax.lax.Precision.DEFAULT,
        )
        x = y_ref[...] + temb
        jn = jnp.full((d, d), 1.0 / d, jnp.float32)
        mean = jax.lax.dot(x, jn, precision=jax.lax.Precision.DEFAULT)
        c = x - mean
        var = jax.lax.dot(c * c, jn, precision=jax.lax.Precision.DEFAULT)
        r = jax.lax.rsqrt(var + EPS)
        o_ref[...] = (c * r) * g_ref[...] + b_ref[...]

    in_specs = [
        pl.BlockSpec((1, 1, tb), lambda i: (i, 0, 0)),
        pl.BlockSpec((tb, d), lambda i: (i, 0)),
        pl.BlockSpec((d, d), lambda i: (0, 0)),
        pl.BlockSpec((1, d), lambda i: (0, 0)),
        pl.BlockSpec((1, d), lambda i: (0, 0)),
    ]
    args = [tids3_k, y_k, tt_pad, gamma2d, beta2d]
    aliases = {}
    if buf is not None:
        in_specs = [pl.BlockSpec(memory_space=pl.ANY)] + in_specs
        args = [buf] + args
        aliases = {0: 0}

    return pl.pallas_call(
        body,
        grid=(slab_blocks,),
        in_specs=in_specs,
        out_specs=pl.BlockSpec((tb, d), lambda i: (block0 + i, 0)),
        out_shape=jax.ShapeDtypeStruct((n_tokens, d), jnp.float32),
        input_output_aliases=aliases,
    )(*args)


K_SLABS = 5


def kernel(input_ids, type_ids, input_table, type_table, ln_gamma, ln_beta):
    b, s = input_ids.shape
    d = input_table.shape[1]
    n_tokens = b * s
    tb = 4096

    slab = n_tokens // K_SLABS
    idx2d = input_ids.reshape(n_tokens // IROW, IROW)
    tids3 = type_ids.reshape(n_tokens // tb, 1, tb)

    tv = type_table.shape[0]
    tt_pad = jnp.zeros((d, d), jnp.float32).at[:tv].set(type_table)
    g2 = ln_gamma.reshape(1, d)
    b2 = ln_beta.reshape(1, d)

    srows = slab // IROW
    sblocks = slab // tb
    ys = [
        _sc_gather(input_table,
                   lax.slice_in_dim(idx2d, k * srows, (k + 1) * srows),
                   slab, d)
        for k in range(K_SLABS)
    ]
    buf = None
    for k in range(K_SLABS):
        buf = _tc_type_ln_slab(
            buf, ys[k],
            lax.slice_in_dim(tids3, k * sblocks, (k + 1) * sblocks),
            tt_pad, g2, b2, tb, d, n_tokens, k * sblocks,
        )
    return buf.reshape(b, s, d)
